# Initial kernel scaffold; baseline (speedup 1.0000x reference)
import jax
import jax.numpy as jnp
from jax.experimental import pallas as pl


def kernel(x, pos, batch, params):
    raise NotImplementedError("write your pallas kernel here")


# speedup vs baseline: 20.7223x; 20.7223x over previous
import dataclasses
import functools

import jax
import jax.numpy as jnp
import numpy as np
from jax import lax
from jax.experimental import pallas as pl
from jax.experimental.pallas import tpu as pltpu
from jax.experimental.pallas import tpu_sc as plsc

B = 8
N = 2048
M1 = 1024
M2 = 256
MAXN = 64
R1SQ = 0.2 * 0.2
R2SQ = 0.4 * 0.4
INVSTD = float(1.0 / np.sqrt(1.0 + 1e-5))
TD1 = 16
TD2 = 80
NEG = -1e30
F32 = jnp.float32
I32 = jnp.int32


def _fps_body(posx_ref, posy_ref, p1x_ref, p1y_ref, p2x_ref, p2y_ref):
    px = posx_ref[...]
    py = posy_ref[...]

    def run_level(sx, sy, n_src, n_smp):
        il = lax.broadcasted_iota(I32, (B, n_src), 1)
        ol = lax.broadcasted_iota(I32, (B, n_smp), 1)

        def body(i, st):
            dmin, ox, oy, lx, ly = st
            d = (sx - lx) ** 2 + (sy - ly) ** 2
            dmin = jnp.minimum(dmin, d)
            mx = jnp.max(dmin, axis=1, keepdims=True)
            am = jnp.min(jnp.where(dmin == mx, il, n_src), axis=1,
                         keepdims=True)
            sel = il == am
            lx = jnp.max(jnp.where(sel, sx, -jnp.inf), axis=1, keepdims=True)
            ly = jnp.max(jnp.where(sel, sy, -jnp.inf), axis=1, keepdims=True)
            ox = jnp.where(ol == i, lx, ox)
            oy = jnp.where(ol == i, ly, oy)
            return dmin, ox, oy, lx, ly

        st0 = (jnp.full((B, n_src), jnp.inf, F32),
               jnp.broadcast_to(sx[:, :1], (B, n_smp)),
               jnp.broadcast_to(sy[:, :1], (B, n_smp)),
               sx[:, :1], sy[:, :1])
        _, ox, oy, _, _ = lax.fori_loop(1, n_smp, body, st0)
        return ox, oy

    p1x, p1y = run_level(px, py, N, M1)
    p2x, p2y = run_level(p1x, p1y, M1, M2)
    p1x_ref[...] = p1x
    p1y_ref[...] = p1y
    p2x_ref[...] = p2x
    p2y_ref[...] = p2y


def _fps_call(posx, posy):
    out = jax.ShapeDtypeStruct
    return pl.pallas_call(
        _fps_body,
        out_shape=[out((B, M1), F32), out((B, M1), F32),
                   out((B, M2), F32), out((B, M2), F32)],
    )(posx, posy)


def _sc_compiler_params():
    cp = pltpu.CompilerParams()
    fields = pltpu.CompilerParams.__dataclass_fields__
    if "needs_layout_passes" in fields:
        cp = dataclasses.replace(cp, needs_layout_passes=False)
    if "use_tc_tiling_on_sc" in fields:
        cp = dataclasses.replace(cp, use_tc_tiling_on_sc=False)
    return cp


def _select(nq, nchunks, r2, sx_ref, sy_ref, qx_ref, qy_ref, cand_ref,
            d2c_ref, nbr_ref, cntl_ref, base):
    lane = lax.iota(I32, 16)
    inf16 = jnp.full((16,), jnp.inf, F32)
    big = jnp.int32(1 << 30)

    def per_query(m, _):
        msplat = jnp.full((16,), m, I32)
        qxs = plsc.load_gather(qx_ref, [msplat])
        qys = plsc.load_gather(qy_ref, [msplat])

        def chunk_body(k, cnt):
            j0 = k * 16
            dx = sx_ref[pl.ds(j0, 16)] - qxs
            dy = sy_ref[pl.ds(j0, 16)] - qys
            d2 = dx * dx + dy * dy
            msk = d2 <= r2
            plsc.store_compressed(cand_ref.at[pl.ds(cnt, 16)], lane + j0,
                                  mask=msk)
            return cnt + jnp.sum(msk.astype(I32))

        cnt = lax.fori_loop(0, nchunks, chunk_body, jnp.int32(0))
        ncap = jnp.minimum(cnt, MAXN)
        plsc.store_scatter(cntl_ref, [jnp.full((16,), m, I32)],
                           jnp.full((16,), ncap, I32), mask=lane == 0)

        def light(_):
            for t in range(4):
                v = cand_ref[pl.ds(t * 16, 16)] + base
                nbr_ref[pl.ds(m * MAXN + t * 16, 16)] = v
            return 0

        def heavy(_):
            cc = (cnt + 15) // 16

            def mkd2(k, _):
                j0 = k * 16
                cv = cand_ref[pl.ds(j0, 16)]
                gx = plsc.load_gather(sx_ref, [cv])
                gy = plsc.load_gather(sy_ref, [cv])
                dx = gx - qxs
                dy = gy - qys
                d2 = dx * dx + dy * dy
                valid = (lane + j0) < cnt
                d2c_ref[pl.ds(j0, 16)] = jnp.where(valid, d2, jnp.inf)
                return 0

            lax.fori_loop(0, cc, mkd2, 0)

            def extract(t, _):
                def mn(k, mv):
                    return jnp.minimum(mv, d2c_ref[pl.ds(k * 16, 16)])

                m0 = jnp.min(lax.fori_loop(0, cc, mn, inf16))

                def fi(k, bi):
                    j0 = k * 16
                    cv = cand_ref[pl.ds(j0, 16)]
                    d2v = d2c_ref[pl.ds(j0, 16)]
                    return jnp.minimum(bi, jnp.min(
                        jnp.where(d2v == m0, cv, big)))

                bi = lax.fori_loop(0, cc, fi, big)
                plsc.store_scatter(nbr_ref,
                                   [jnp.full((16,), m * MAXN + t, I32)],
                                   jnp.full((16,), bi + base, I32),
                                   mask=lane == 0)

                def inv(k, _):
                    j0 = k * 16
                    cv = cand_ref[pl.ds(j0, 16)]
                    d2v = d2c_ref[pl.ds(j0, 16)]
                    d2c_ref[pl.ds(j0, 16)] = jnp.where(cv == bi, jnp.inf, d2v)
                    return 0

                lax.fori_loop(0, cc, inv, 0)
                return 0

            lax.fori_loop(0, MAXN, extract, 0)
            return 0

        lax.cond(cnt > MAXN, heavy, light, 0)
        return 0

    lax.fori_loop(0, nq, per_query, 0)


def _s1_body(posx_h, posy_h, p1x_h, p1y_h, p2x_h, p2y_h, tab1_h,
             g1_h, cnt1_h, nbr2_h, cnt2_h,
             sx, sy, s1x, s1y, qx, qy, q2x, q2y, cand, d2c,
             nbr1, cnt1l, nbr2l, cnt2l, buf0, buf1, sem0, sem1):
    w = lax.axis_index("s") * 2 + lax.axis_index("c")
    c = w // 4
    part = w % 4
    def zinit(k, _):
        cand[pl.ds(k * 16, 16)] = jnp.zeros((16,), I32)
        return 0
    lax.fori_loop(0, 129, zinit, 0)

    pltpu.sync_copy(posx_h.at[c], sx)
    pltpu.sync_copy(posy_h.at[c], sy)
    pltpu.sync_copy(p1x_h.at[c], s1x)
    pltpu.sync_copy(p1y_h.at[c], s1y)
    qoff = part * 256
    pltpu.sync_copy(p1x_h.at[c, pl.ds(qoff, 256)], qx)
    pltpu.sync_copy(p1y_h.at[c, pl.ds(qoff, 256)], qy)
    q2off = part * 64
    pltpu.sync_copy(p2x_h.at[c, pl.ds(q2off, 64)], q2x)
    pltpu.sync_copy(p2y_h.at[c, pl.ds(q2off, 64)], q2y)

    _select(256, N // 16, R1SQ, sx, sy, qx, qy, cand, d2c,
            nbr1, cnt1l, c * N)
    gq0 = c * M1 + qoff
    pltpu.sync_copy(cnt1l, cnt1_h.at[pl.ds(gq0, 256)])

    _select(64, M1 // 16, R2SQ, s1x, s1y, q2x, q2y, cand, d2c,
            nbr2l, cnt2l, c * M1)
    gq20 = c * M2 + q2off
    pltpu.sync_copy(nbr2l, nbr2_h.at[pl.ds(gq20 * MAXN, 64 * MAXN)])
    pltpu.sync_copy(cnt2l, cnt2_h.at[pl.ds(gq20, 64)])

    row0 = gq0 * MAXN
    bufs = (buf0, buf1)
    sems = (sem0, sem1)
    copies = []
    for ch in range(8):
        b = ch % 2
        cp = pltpu.make_async_copy(
            tab1_h.at[nbr1.at[pl.ds(ch * 2048, 2048)]], bufs[b], sems[b])
        if ch >= 1:
            copies[ch - 1].wait()
        cp.start()
        copies.append(cp)
        if ch >= 1:
            pltpu.sync_copy(bufs[1 - b],
                            g1_h.at[pl.ds(row0 + (ch - 1) * 2048, 2048)])
    copies[7].wait()
    pltpu.sync_copy(bufs[1], g1_h.at[pl.ds(row0 + 7 * 2048, 2048)])


def _s1_call(posx, posy, p1x, p1y, p2x, p2y, table1):
    mesh = plsc.VectorSubcoreMesh(core_axis_name="c", subcore_axis_name="s")
    sds = jax.ShapeDtypeStruct
    vm = pltpu.VMEM
    kern = pl.kernel(
        _s1_body,
        mesh=mesh,
        out_type=[sds((B * M1 * MAXN, TD1), F32), sds((B * M1,), I32),
                  sds((B * M2 * MAXN,), I32), sds((B * M2,), I32)],
        scratch_types=[vm((N,), F32), vm((N,), F32),
                       vm((M1,), F32), vm((M1,), F32),
                       vm((256,), F32), vm((256,), F32),
                       vm((64,), F32), vm((64,), F32),
                       vm((2064,), I32), vm((2064,), F32),
                       vm((256 * MAXN,), I32), vm((256,), I32),
                       vm((64 * MAXN,), I32), vm((64,), I32),
                       vm((2048, TD1), F32), vm((2048, TD1), F32),
                       pltpu.SemaphoreType.DMA, pltpu.SemaphoreType.DMA],
        compiler_params=_sc_compiler_params(),
    )
    return kern(posx, posy, p1x, p1y, p2x, p2y, table1)


def _s2_body(tab2_h, nbr2_h, g2_h, idxv, buf0, buf1, sem0, sem1):
    w = lax.axis_index("s") * 2 + lax.axis_index("c")
    base = w * 4096
    bufs = (buf0, buf1)
    sems = (sem0, sem1)
    pltpu.sync_copy(nbr2_h.at[pl.ds(base, 4096)], idxv)
    copies = []
    for ch in range(8):
        b = ch % 2
        cp = pltpu.make_async_copy(
            tab2_h.at[idxv.at[pl.ds(ch * 512, 512)]], bufs[b], sems[b])
        if ch >= 1:
            copies[ch - 1].wait()
        cp.start()
        copies.append(cp)
        if ch >= 1:
            pltpu.sync_copy(bufs[1 - b],
                            g2_h.at[pl.ds(base + (ch - 1) * 512, 512)])
    copies[7].wait()
    pltpu.sync_copy(bufs[1], g2_h.at[pl.ds(base + 7 * 512, 512)])


def _s2_call(table2, nbr2):
    mesh = plsc.VectorSubcoreMesh(core_axis_name="c", subcore_axis_name="s")
    sds = jax.ShapeDtypeStruct
    vm = pltpu.VMEM
    kern = pl.kernel(
        _s2_body,
        mesh=mesh,
        out_type=sds((B * M2 * MAXN, TD2), F32),
        scratch_types=[vm((4096,), I32),
                       vm((512, TD2), F32), vm((512, TD2), F32),
                       pltpu.SemaphoreType.DMA, pltpu.SemaphoreType.DMA],
        compiler_params=_sc_compiler_params(),
    )
    return kern(table2, nbr2)


def _sa1_body(g1_ref, px_ref, py_ref, cnt_ref, w1c_ref, w1px_ref, w1py_ref,
              b1_ref, ga1_ref, be1_ref, w2t_ref, b2_ref, t2_ref):
    G = g1_ref[...]
    H = jnp.dot(G, w1c_ref[...], preferred_element_type=F32)
    px = px_ref[0, 0, :]
    py = py_ref[0, 0, :]
    ty = b1_ref[...] - px[:, None] * w1px_ref[...] - py[:, None] * w1py_ref[...]
    H = H.reshape(256, MAXN, 32) + ty[:, None, :]
    H = jnp.maximum(H * ga1_ref[...][None] + be1_ref[...][None], 0.0)
    M = jnp.dot(H.reshape(256 * MAXN, 32), w2t_ref[...],
                preferred_element_type=F32) + b2_ref[...]
    M = M.reshape(256, MAXN, 64)
    slot = lax.broadcasted_iota(I32, (256, MAXN, 1), 1)
    cnt = cnt_ref[0, 0, :]
    M = jnp.where(slot < cnt[:, None, None], M, NEG)
    o = jnp.max(M, axis=1)
    o = jnp.where(o <= -1e20, 0.0, o)
    T2 = jnp.concatenate(
        [o, px[:, None], py[:, None], jnp.zeros((256, TD2 - 66), F32)], axis=1)
    t2_ref[...] = T2


def _sa1_call(g1, p1x, p1y, cnt1, w1c, w1px, w1py, b1, ga1, be1, w2t, b2):
    nblk = B * M1 // 256
    full = lambda shape: pl.BlockSpec(shape, lambda i: tuple(0 for _ in shape))
    return pl.pallas_call(
        _sa1_body,
        grid=(nblk,),
        in_specs=[
            pl.BlockSpec((256 * MAXN, TD1), lambda i: (i, 0)),
            pl.BlockSpec((1, 1, 256), lambda i: (i, 0, 0)),
            pl.BlockSpec((1, 1, 256), lambda i: (i, 0, 0)),
            pl.BlockSpec((1, 1, 256), lambda i: (i, 0, 0)),
            full((TD1, 32)), full((1, 32)), full((1, 32)), full((1, 32)),
            full((1, 32)), full((1, 32)), full((32, 64)), full((1, 64)),
        ],
        out_specs=pl.BlockSpec((256, TD2), lambda i: (i, 0)),
        out_shape=jax.ShapeDtypeStruct((B * M1, TD2), F32),
    )(g1, p1x.reshape(nblk, 1, 256), p1y.reshape(nblk, 1, 256),
      cnt1.reshape(nblk, 1, 256), w1c, w1px, w1py, b1, ga1, be1, w2t, b2)


def _sa2_body(g2_ref, px_ref, py_ref, cnt_ref, w2c_ref, w2px_ref, w2py_ref,
              b21_ref, ga2_ref, be2_ref, w22_ref, b22_ref,
              w3a_ref, b3a_ref, ga3_ref, be3_ref, w3b_ref, b3b_ref,
              hw1_ref, hb1_ref, hw2_ref, hb2_ref, out_ref):
    G = g2_ref[...]
    H = jnp.dot(G, w2c_ref[...], preferred_element_type=F32)
    px = px_ref[0, 0, :]
    py = py_ref[0, 0, :]
    ty = b21_ref[...] - px[:, None] * w2px_ref[...] - py[:, None] * w2py_ref[...]
    H = H.reshape(M2, MAXN, 64) + ty[:, None, :]
    H = jnp.maximum(H * ga2_ref[...][None] + be2_ref[...][None], 0.0)
    M = jnp.dot(H.reshape(M2 * MAXN, 64), w22_ref[...],
                preferred_element_type=F32) + b22_ref[...]
    M = M.reshape(M2, MAXN, 128)
    slot = lax.broadcasted_iota(I32, (M2, MAXN, 1), 1)
    cnt = cnt_ref[0, 0, :]
    M = jnp.where(slot < cnt[:, None, None], M, NEG)
    x2 = jnp.max(M, axis=1)
    x2 = jnp.where(x2 <= -1e20, 0.0, x2)
    t = jnp.concatenate([x2, px[:, None], py[:, None]], axis=1)
    h = jnp.dot(t, w3a_ref[...], preferred_element_type=F32) + b3a_ref[...]
    h = jnp.maximum(h * ga3_ref[...] + be3_ref[...], 0.0)
    h = jnp.dot(h, w3b_ref[...], preferred_element_type=F32) + b3b_ref[...]
    g = jnp.max(h, axis=0, keepdims=True)
    z = jnp.maximum(jnp.dot(g, hw1_ref[...],
                            preferred_element_type=F32) + hb1_ref[...], 0.0)
    o = jnp.dot(z, hw2_ref[...], preferred_element_type=F32) + hb2_ref[...]
    out_ref[...] = o.reshape(1, 1, 1024)


def _sa2_call(g2, p2x, p2y, cnt2, w2c, w2px, w2py, b21, ga2, be2, w22, b22,
              w3a, b3a, ga3, be3, w3b, b3b, hw1, hb1, hw2, hb2):
    full = lambda shape: pl.BlockSpec(shape, lambda i: tuple(0 for _ in shape))
    out = pl.pallas_call(
        _sa2_body,
        grid=(B,),
        in_specs=[
            pl.BlockSpec((M2 * MAXN, TD2), lambda i: (i, 0)),
            pl.BlockSpec((1, 1, M2), lambda i: (i, 0, 0)),
            pl.BlockSpec((1, 1, M2), lambda i: (i, 0, 0)),
            pl.BlockSpec((1, 1, M2), lambda i: (i, 0, 0)),
            full((TD2, 64)), full((1, 64)), full((1, 64)), full((1, 64)),
            full((1, 64)), full((1, 64)), full((64, 128)), full((1, 128)),
            full((130, 128)), full((1, 128)), full((1, 128)), full((1, 128)),
            full((128, 256)), full((1, 256)),
            full((256, 256)), full((1, 256)), full((256, 1024)),
            full((1, 1024)),
        ],
        out_specs=pl.BlockSpec((1, 1, 1024), lambda i: (i, 0, 0)),
        out_shape=jax.ShapeDtypeStruct((B, 1, 1024), F32),
    )(g2, p2x.reshape(B, 1, M2), p2y.reshape(B, 1, M2),
      cnt2.reshape(B, 1, M2), w2c, w2px, w2py, b21, ga2, be2, w22, b22,
      w3a, b3a, ga3, be3, w3b, b3b, hw1, hb1, hw2, hb2)
    return out


def kernel(x, pos, batch, params):
    p = params
    posb = pos.reshape(B, N, 2)
    posx = posb[:, :, 0]
    posy = posb[:, :, 1]
    table1 = jnp.concatenate(
        [x, pos, jnp.zeros((B * N, TD1 - 7), F32)], axis=1)

    p1x, p1y, p2x, p2y = _fps_call(posx, posy)
    g1, cnt1, nbr2, cnt2 = _s1_call(posx, posy, p1x, p1y, p2x, p2y, table1)

    w1 = p['sa1_W1']
    w1c = jnp.concatenate([w1.T, jnp.zeros((TD1 - 7, 32), F32)], axis=0)
    w1px = w1[:, 5].reshape(1, 32)
    w1py = w1[:, 6].reshape(1, 32)
    table2 = _sa1_call(
        g1, p1x, p1y, cnt1, w1c, w1px, w1py,
        p['sa1_b1'].reshape(1, 32), (p['sa1_g1'] * INVSTD).reshape(1, 32),
        p['sa1_be1'].reshape(1, 32), p['sa1_W2'].T, p['sa1_b2'].reshape(1, 64))

    g2 = _s2_call(table2, nbr2)

    w2 = p['sa2_W1']
    w2c = jnp.concatenate([w2.T, jnp.zeros((TD2 - 66, 64), F32)], axis=0)
    w2px = w2[:, 64].reshape(1, 64)
    w2py = w2[:, 65].reshape(1, 64)
    out = _sa2_call(
        g2, p2x, p2y, cnt2, w2c, w2px, w2py,
        p['sa2_b1'].reshape(1, 64), (p['sa2_g1'] * INVSTD).reshape(1, 64),
        p['sa2_be1'].reshape(1, 64), p['sa2_W2'].T, p['sa2_b2'].reshape(1, 128),
        p['sa3_W1'].T, p['sa3_b1'].reshape(1, 128),
        (p['sa3_g1'] * INVSTD).reshape(1, 128), p['sa3_be1'].reshape(1, 128),
        p['sa3_W2'].T, p['sa3_b2'].reshape(1, 256),
        p['head_W1'].T, p['head_b1'].reshape(1, 256),
        p['head_W2'].T, p['head_b2'].reshape(1, 1024))
    return out.reshape(B, 1024)
